# Initial kernel scaffold; baseline (speedup 1.0000x reference)
import jax
import jax.numpy as jnp
from jax.experimental import pallas as pl


def kernel(H_0, Z, block_id, batch_id, edges, params):
    raise NotImplementedError("write your pallas kernel here")


# speedup vs baseline: 2.9230x; 2.9230x over previous
import functools

import jax
import jax.numpy as jnp
from jax import lax
from jax.experimental import pallas as pl
from jax.experimental.pallas import tpu as pltpu
from jax.experimental.pallas import tpu_sc as plsc

_NC, _NS = 2, 16
_NW = _NC * _NS


def _sc_mesh():
    return plsc.VectorSubcoreMesh(
        core_axis_name="c", subcore_axis_name="s",
        num_cores=_NC, num_subcores=_NS)


def _sc_gather2(table, idx_a, idx_b, chunk, name):
    M = idx_a.shape[0]
    D = table.shape[1]
    per_w = M // _NW
    n_ch = per_w // chunk
    assert per_w % chunk == 0 and M % _NW == 0

    @functools.partial(
        pl.kernel, mesh=_sc_mesh(),
        out_type=(jax.ShapeDtypeStruct((M, D), table.dtype),
                  jax.ShapeDtypeStruct((M, D), table.dtype)),
        scratch_types=[pltpu.VMEM((chunk,), jnp.int32),
                       pltpu.VMEM((chunk, D), table.dtype),
                       pltpu.VMEM((chunk,), jnp.int32),
                       pltpu.VMEM((chunk, D), table.dtype),
                       pltpu.SemaphoreType.DMA,
                       pltpu.SemaphoreType.DMA],
        name=name)
    def k(table_h, ia_h, ib_h, oa_h, ob_h, ia_v, ra_v, ib_v, rb_v, sa, sb):
        wid = lax.axis_index("s") * _NC + lax.axis_index("c")
        base = wid * per_w

        def body(j, carry):
            off = pl.multiple_of(base + j * chunk, 8)
            pltpu.sync_copy(ia_h.at[pl.ds(off, chunk)], ia_v)
            pltpu.sync_copy(ib_h.at[pl.ds(off, chunk)], ib_v)
            ca = pltpu.async_copy(table_h.at[ia_v], ra_v, sa)
            cb = pltpu.async_copy(table_h.at[ib_v], rb_v, sb)
            ca.wait()
            pltpu.sync_copy(ra_v, oa_h.at[pl.ds(off, chunk)])
            cb.wait()
            pltpu.sync_copy(rb_v, ob_h.at[pl.ds(off, chunk)])
            return carry

        lax.fori_loop(0, n_ch, body, 0)

    return k(table, idx_a, idx_b)


def _sc_gather1(table, idx, chunk, name):
    M = idx.shape[0]
    D = table.shape[1]
    per_w = M // _NW
    n_ch = per_w // chunk
    assert per_w % chunk == 0 and M % _NW == 0

    @functools.partial(
        pl.kernel, mesh=_sc_mesh(),
        out_type=jax.ShapeDtypeStruct((M, D), table.dtype),
        scratch_types=[pltpu.VMEM((chunk,), jnp.int32),
                       pltpu.VMEM((chunk, D), table.dtype),
                       pltpu.SemaphoreType.DMA],
        name=name)
    def k(table_h, idx_h, out_h, idx_v, rows_v, sem):
        wid = lax.axis_index("s") * _NC + lax.axis_index("c")
        base = wid * per_w

        def body(j, carry):
            off = pl.multiple_of(base + j * chunk, 8)
            pltpu.sync_copy(idx_h.at[pl.ds(off, chunk)], idx_v)
            pltpu.async_copy(table_h.at[idx_v], rows_v, sem).wait()
            pltpu.sync_copy(rows_v, out_h.at[pl.ds(off, chunk)])
            return carry

        lax.fori_loop(0, n_ch, body, 0)

    return k(table, idx)


def _zslice(nrows):
    for k in range(_NS, 0, -1):
        if nrows % k == 0 and (nrows // k) % 8 == 0:
            return k, nrows // k
    raise ValueError(nrows)


def _sc_segsum(vals, idx, nrows, chunk, zeros, name):
    M, D = vals.shape
    per_w = M // _NW
    n_ch = per_w // chunk
    n_zt, rows_t = _zslice(nrows)
    assert per_w % chunk == 0

    @functools.partial(
        pl.kernel, mesh=_sc_mesh(),
        out_type=jax.ShapeDtypeStruct((2 * nrows, D), vals.dtype),
        scratch_types=[pltpu.VMEM((chunk,), jnp.int32),
                       pltpu.VMEM((chunk, D), vals.dtype),
                       pltpu.VMEM_SHARED((nrows, D), vals.dtype),
                       pltpu.SemaphoreType.DMA],
        name=name)
    def k(vals_h, idx_h, zeros_h, out_h, idx_v, buf_v, acc_s, sem):
        c = lax.axis_index("c")
        s = lax.axis_index("s")
        wid = s * _NC + c
        z_off = pl.multiple_of(s * rows_t, 8)

        @pl.when(s < n_zt)
        def _():
            pltpu.sync_copy(zeros_h.at[pl.ds(z_off, rows_t)],
                            acc_s.at[pl.ds(z_off, rows_t)])

        plsc.subcore_barrier()
        base = wid * per_w

        def body(j, carry):
            off = pl.multiple_of(base + j * chunk, 8)
            pltpu.sync_copy(idx_h.at[pl.ds(off, chunk)], idx_v)
            pltpu.sync_copy(vals_h.at[pl.ds(off, chunk)], buf_v)
            pltpu.sync_copy(buf_v, acc_s.at[idx_v], add=True)
            return carry

        lax.fori_loop(0, n_ch, body, 0)
        plsc.subcore_barrier()

        @pl.when(s < n_zt)
        def _():
            pltpu.sync_copy(acc_s.at[pl.ds(z_off, rows_t)],
                            out_h.at[pl.ds(c * nrows + z_off, rows_t)])

    return k(vals, idx, zeros)


def _sc_gather_segsum(table, src, dst, nrows, chunk, zeros, name):
    M = src.shape[0]
    D = table.shape[1]
    per_w = M // _NW
    n_ch = per_w // chunk
    n_zt, rows_t = _zslice(nrows)
    assert per_w % chunk == 0

    @functools.partial(
        pl.kernel, mesh=_sc_mesh(),
        out_type=jax.ShapeDtypeStruct((2 * nrows, D), table.dtype),
        scratch_types=[pltpu.VMEM((chunk,), jnp.int32),
                       pltpu.VMEM((chunk,), jnp.int32),
                       pltpu.VMEM((chunk, D), table.dtype),
                       pltpu.VMEM_SHARED((nrows, D), table.dtype),
                       pltpu.SemaphoreType.DMA],
        name=name)
    def k(table_h, src_h, dst_h, zeros_h, out_h, si_v, di_v, buf_v, acc_s, sem):
        c = lax.axis_index("c")
        s = lax.axis_index("s")
        wid = s * _NC + c
        z_off = pl.multiple_of(s * rows_t, 8)

        @pl.when(s < n_zt)
        def _():
            pltpu.sync_copy(zeros_h.at[pl.ds(z_off, rows_t)],
                            acc_s.at[pl.ds(z_off, rows_t)])

        plsc.subcore_barrier()
        base = wid * per_w

        def body(j, carry):
            off = pl.multiple_of(base + j * chunk, 8)
            pltpu.sync_copy(src_h.at[pl.ds(off, chunk)], si_v)
            pltpu.sync_copy(dst_h.at[pl.ds(off, chunk)], di_v)
            pltpu.async_copy(table_h.at[si_v], buf_v, sem).wait()
            pltpu.sync_copy(buf_v, acc_s.at[di_v], add=True)
            return carry

        lax.fori_loop(0, n_ch, body, 0)
        plsc.subcore_barrier()

        @pl.when(s < n_zt)
        def _():
            pltpu.sync_copy(acc_s.at[pl.ds(z_off, rows_t)],
                            out_h.at[pl.ds(c * nrows + z_off, rows_t)])

    return k(table, src, dst, zeros)


def _silu(x):
    return x * (1.0 / (1.0 + jnp.exp(-x)))


def _sigm(x):
    return 1.0 / (1.0 + jnp.exp(-x))


def _lrelu(x):
    return jnp.where(x >= 0, x, 0.01 * x)


def _mm(a, b):
    return jax.lax.dot(a, b, precision=jax.lax.Precision.DEFAULT)


def _stats_contrib(t):
    s1 = jnp.sum(t, axis=0, keepdims=True)
    s2 = jnp.sum(t * t, axis=0, keepdims=True)
    h = t.shape[1]
    r = lax.broadcasted_iota(jnp.int32, (8, h), 0)
    return jnp.where(r == 0, jnp.broadcast_to(s1, (8, h)),
                     jnp.where(r == 1, jnp.broadcast_to(s2, (8, h)), 0.0))


def _accum_stats(st_ref, t):
    i = pl.program_id(0)
    contrib = _stats_contrib(t)

    @pl.when(i == 0)
    def _():
        st_ref[...] = contrib

    @pl.when(i > 0)
    def _():
        st_ref[...] += contrib


def _bn_apply(t, st, g, b, n):
    mean = st[0:1, :] / n
    var = st[1:2, :] / n - mean * mean
    return g * (t - mean) / jnp.sqrt(var + 1e-5) + b


def _row_spec(rb, d):
    return pl.BlockSpec((rb, d), lambda i: (i, 0))


def _part_specs(rb, d, nrows):
    off = nrows // rb
    return [pl.BlockSpec((rb, d), lambda i: (i, 0)),
            pl.BlockSpec((rb, d), lambda i, _o=off: (i + _o, 0))]


def _full_spec(shape):
    return pl.BlockSpec(shape, lambda i: tuple(0 for _ in shape))


def kernel(H_0, Z, block_id, batch_id, edges, params):
    p = params
    N, Hd = H_0.shape
    E = edges.shape[1]
    NB = batch_id.shape[0]
    BS = 16
    f32 = jnp.float32

    src = edges[0].astype(jnp.int32)
    dst = edges[1].astype(jnp.int32)
    blk_id = block_id.astype(jnp.int32)
    bat_id = batch_id.astype(jnp.int32)

    RBN = 1000
    BE = 3200
    GN = N // RBN
    GE = E // BE
    CH = 80

    pos = Z.reshape(N, 3)
    pos128 = jnp.pad(pos, ((0, 0), (0, Hd - 3)))

    zeros_n128 = jnp.zeros((N, Hd), f32)

    def b_row(name):
        return p[name + '_b'].reshape(1, -1)

    def k_enc(h0_ref, we, be, wn, bn, xpsc_ref, xraw_ref):
        h0 = h0_ref[...]
        xl = _silu(_mm(h0, we[...]) + be[...])
        nrm = jnp.sqrt(jnp.sum(xl * xl, axis=1, keepdims=True))
        xpsc_ref[...] = xl / jnp.maximum(nrm, 1e-12) * 1.8
        xraw_ref[...] = _silu(_mm(h0, wn[...]) + bn[...])

    x_psc, x_raw = pl.pallas_call(
        k_enc, grid=(GN,),
        in_specs=[_row_spec(RBN, Hd), _full_spec((Hd, Hd)), _full_spec((1, Hd)),
                  _full_spec((Hd, Hd)), _full_spec((1, Hd))],
        out_specs=[_row_spec(RBN, Hd), _row_spec(RBN, Hd)],
        out_shape=[jax.ShapeDtypeStruct((N, Hd), f32),
                   jax.ShapeDtypeStruct((N, Hd), f32)],
        name="tc_enc")(H_0, p['enc_lin_W'], b_row('enc_lin'),
                       p['lin_node_W'], b_row('lin_node'))

    ps128, pd128 = _sc_gather2(pos128, src, dst, CH, "sc_gather_pos")

    def k_dist(ps_ref, pd_ref, out_ref):
        d = ps_ref[...] - pd_ref[...]
        d2 = jnp.sum(d * d, axis=1, keepdims=True)
        out_ref[...] = jnp.broadcast_to(jnp.sqrt(d2 + 1e-12), out_ref.shape)

    dist16 = pl.pallas_call(
        k_dist, grid=(GE,),
        in_specs=[_row_spec(BE, Hd), _row_spec(BE, Hd)],
        out_specs=_row_spec(BE, Hd),
        out_shape=jax.ShapeDtypeStruct((E, Hd), f32),
        name="tc_dist")(ps128, pd128)

    deg_parts = _sc_segsum(dist16, dst, N, CH, zeros_n128, "sc_seg_deg")

    def k_dinv(p0_ref, p1_ref, out_ref):
        deg = p0_ref[...] + p1_ref[...] + 1.0
        out_ref[...] = 1.0 / jnp.sqrt(deg)

    dinv16 = pl.pallas_call(
        k_dinv, grid=(GN,),
        in_specs=_part_specs(RBN, Hd, N),
        out_specs=_row_spec(RBN, Hd),
        out_shape=jax.ShapeDtypeStruct((N, Hd), f32),
        name="tc_dinv")(deg_parts, deg_parts)

    dvs16, dvd16 = _sc_gather2(dinv16, src, dst, CH, "sc_gather_dinv")
    xps = _sc_gather1(x_psc, src, CH, "sc_gather_xpsc")

    def k_scale(dvs_ref, d16_ref, dvd_ref, xps_ref, out_ref):
        nrm = dvs_ref[...][:, :1] * d16_ref[...][:, :1] * dvd_ref[...][:, :1]
        out_ref[...] = nrm * xps_ref[...]

    scaled = pl.pallas_call(
        k_scale, grid=(GE,),
        in_specs=[_row_spec(BE, Hd), _row_spec(BE, Hd), _row_spec(BE, Hd),
                  _row_spec(BE, Hd)],
        out_specs=_row_spec(BE, Hd),
        out_shape=jax.ShapeDtypeStruct((E, Hd), f32),
        name="tc_appnp_scale")(dvs16, dist16, dvd16, xps)

    h_parts = _sc_segsum(scaled, dst, N, CH, zeros_n128, "sc_seg_appnp")

    def k_enc2(xpsc_ref, hp0_ref, hp1_ref, dv_ref, xraw_ref, w, b,
               t_ref, st_ref):
        xp = xpsc_ref[...]
        dv = dv_ref[...][:, :1]
        h = hp0_ref[...] + hp1_ref[...] + xp * (dv * dv)
        xint = 0.1 * xp + 0.9 * h
        t = _lrelu(_mm(xint + xraw_ref[...], w[...]) + b[...])
        t_ref[...] = t
        _accum_stats(st_ref, t)

    t_enc, st_enc = pl.pallas_call(
        k_enc2, grid=(GN,),
        in_specs=[_row_spec(RBN, Hd)] + _part_specs(RBN, Hd, N)
        + [_row_spec(RBN, Hd), _row_spec(RBN, Hd),
           _full_spec((Hd, Hd)), _full_spec((1, Hd))],
        out_specs=[_row_spec(RBN, Hd), _full_spec((8, Hd))],
        out_shape=[jax.ShapeDtypeStruct((N, Hd), f32),
                   jax.ShapeDtypeStruct((8, Hd), f32)],
        name="tc_enc2")(x_psc, h_parts, h_parts, dinv16, x_raw,
                        p['mlp_enc_W'], b_row('mlp_enc'))

    def k_bn(t_ref, st, g, b, out_ref):
        out_ref[...] = _bn_apply(t_ref[...], st[...], g[...], b[...], float(N))

    def bn_apply_call(t, st, gname):
        return pl.pallas_call(
            k_bn, grid=(GN,),
            in_specs=[_row_spec(RBN, t.shape[1]), _full_spec((8, t.shape[1])),
                      _full_spec((1, t.shape[1])), _full_spec((1, t.shape[1]))],
            out_specs=_row_spec(RBN, t.shape[1]),
            out_shape=jax.ShapeDtypeStruct(t.shape, f32),
            name="tc_bn_" + gname)(t, st, p[gname + '_g'].reshape(1, -1),
                                   p[gname + '_B'].reshape(1, -1))

    x = bn_apply_call(t_enc, st_enc, 'mlp_enc_bn')

    xs, xd = _sc_gather2(x, src, dst, CH, "sc_gather_x")

    eu_i = p['eu_inter_W']
    eu_a = p['eu_intra_W']

    def k_edge(xs_ref, xd_ref, d16_ref,
               wrbf_i, brbf_i, w1i, w2i, w3i, bi,
               wrbf_a, brbf_a, w1a, w2a, w3a, ba,
               mi_ref, ma_ref):
        xse = xs_ref[...]
        xde = xd_ref[...]
        d16 = d16_ref[...][:, :16]
        mu = lax.broadcasted_iota(jnp.int32, d16.shape, 1).astype(f32) * (6.0 / 15.0)
        zz = (d16 - mu) / 0.375
        rbf = jnp.exp(-(zz * zz))

        ea = _sigm(_mm(rbf, wrbf_i[...]) + brbf_i[...])
        u = _silu(_mm(xse, w1i[...]) + _mm(xde, w2i[...]) + _mm(ea, w3i[...]) + bi[...])
        mi_ref[...] = jnp.maximum(xse + u, 0.0)

        ea = _sigm(_mm(rbf, wrbf_a[...]) + brbf_a[...])
        u = _silu(_mm(xse, w1a[...]) + _mm(xde, w2a[...]) + _mm(ea, w3a[...]) + ba[...])
        ma_ref[...] = jnp.maximum(xse + u, 0.0)

    msg_i, msg_a = pl.pallas_call(
        k_edge, grid=(GE,),
        in_specs=[_row_spec(BE, Hd), _row_spec(BE, Hd), _row_spec(BE, Hd),
                  _full_spec((16, Hd)), _full_spec((1, Hd)),
                  _full_spec((Hd, Hd)), _full_spec((Hd, Hd)),
                  _full_spec((Hd, Hd)), _full_spec((1, Hd)),
                  _full_spec((16, Hd)), _full_spec((1, Hd)),
                  _full_spec((Hd, Hd)), _full_spec((Hd, Hd)),
                  _full_spec((Hd, Hd)), _full_spec((1, Hd))],
        out_specs=[_row_spec(BE, Hd), _row_spec(BE, Hd)],
        out_shape=[jax.ShapeDtypeStruct((E, Hd), f32),
                   jax.ShapeDtypeStruct((E, Hd), f32)],
        name="tc_edge")(xs, xd, dist16,
                        p['ea_inter_W'], b_row('ea_inter'),
                        eu_i[:Hd], eu_i[Hd:2 * Hd], eu_i[2 * Hd:], b_row('eu_inter'),
                        p['ea_intra_W'], b_row('ea_intra'),
                        eu_a[:Hd], eu_a[Hd:2 * Hd], eu_a[2 * Hd:], b_row('eu_intra'))

    seg_i = _sc_segsum(msg_i, dst, N, CH, zeros_n128, "sc_seg_msg_i")
    seg_a = _sc_segsum(msg_a, dst, N, CH, zeros_n128, "sc_seg_msg_a")

    def k_gine(x_ref, p0_ref, p1_ref, w, b, t_ref, st_ref):
        h = x_ref[...] + p0_ref[...] + p1_ref[...]
        t = _lrelu(_mm(h, w[...]) + b[...])
        t_ref[...] = t
        _accum_stats(st_ref, t)

    def gine_call(xin, parts, wname, tag):
        return pl.pallas_call(
            k_gine, grid=(GN,),
            in_specs=[_row_spec(RBN, Hd)] + _part_specs(RBN, Hd, N)
            + [_full_spec((Hd, Hd)), _full_spec((1, Hd))],
            out_specs=[_row_spec(RBN, Hd), _full_spec((8, Hd))],
            out_shape=[jax.ShapeDtypeStruct((N, Hd), f32),
                       jax.ShapeDtypeStruct((8, Hd), f32)],
            name="tc_gine_" + tag)(xin, parts, parts, p[wname + '_W'],
                                   b_row(wname))

    t_i1, st_i1 = gine_call(x, seg_i, 'gin1', 'i1')
    t_a1, st_a1 = gine_call(x, seg_a, 'gin3', 'a1')

    def k_dgnn(h_ref, p0_ref, p1_ref, ws, wn, b, out_ref):
        h = h_ref[...]
        agg = p0_ref[...] + p1_ref[...]
        out_ref[...] = _silu(_mm(h, ws[...]) + _mm(agg, wn[...]) + b[...])

    def dgnn_tower(pref, tag):
        h = x
        for l in range(3):
            parts = _sc_gather_segsum(h, src, dst, N, CH, zeros_n128,
                                      "sc_gseg_%s_%d" % (tag, l))
            bsum = (p['%s_s%d_b' % (pref, l)]
                    + p['%s_n%d_b' % (pref, l)]).reshape(1, -1)
            h = pl.pallas_call(
                k_dgnn, grid=(GN,),
                in_specs=[_row_spec(RBN, Hd)] + _part_specs(RBN, Hd, N)
                + [_full_spec((Hd, Hd)), _full_spec((Hd, Hd)),
                   _full_spec((1, Hd))],
                out_specs=_row_spec(RBN, Hd),
                out_shape=jax.ShapeDtypeStruct((N, Hd), f32),
                name="tc_dgnn_%s_%d" % (tag, l))(
                    h, parts, parts, p['%s_s%d_W' % (pref, l)],
                    p['%s_n%d_W' % (pref, l)], bsum)
        return h

    xi2 = dgnn_tower('dgnn1', 'd1')
    xa2 = dgnn_tower('dgnn3', 'd3')

    segx = _sc_gather_segsum(x, src, dst, N, CH, zeros_n128, "sc_gseg_gin4")
    t_m, st_m = gine_call(x, segx, 'gin4', 'm')

    def k_lin2(t_ref, st, g, bb, x2_ref, wa, wb, b, out_ref):
        x1 = _bn_apply(t_ref[...], st[...], g[...], bb[...], float(N))
        out_ref[...] = _silu(_mm(x1, wa[...]) + _mm(x2_ref[...], wb[...]) + b[...])

    def lin2_call(t, st, bnname, x2, wname, tag):
        w = p[wname + '_W']
        return pl.pallas_call(
            k_lin2, grid=(GN,),
            in_specs=[_row_spec(RBN, Hd), _full_spec((8, Hd)),
                      _full_spec((1, Hd)), _full_spec((1, Hd)),
                      _row_spec(RBN, Hd), _full_spec((Hd, Hd)),
                      _full_spec((Hd, Hd)), _full_spec((1, Hd))],
            out_specs=_row_spec(RBN, Hd),
            out_shape=jax.ShapeDtypeStruct((N, Hd), f32),
            name="tc_lin2_" + tag)(
                t, st, p[bnname + '_g'].reshape(1, -1),
                p[bnname + '_B'].reshape(1, -1), x2,
                w[:Hd], w[Hd:], b_row(wname))

    x_inter2 = lin2_call(t_i1, st_i1, 'gin1_bn', xi2, 'lin1', 'inter')
    x_intra = lin2_call(t_a1, st_a1, 'gin3_bn', xa2, 'lin3', 'intra')

    def k_fc0(xi_ref, xa_ref, tm_ref, stm, gm, bm, w, b, t_ref, st_ref):
        xm = _bn_apply(tm_ref[...], stm[...], gm[...], bm[...], float(N))
        xc = xi_ref[...] + xa_ref[...] + xm
        t = _lrelu(_mm(xc, w[...]) + b[...])
        t_ref[...] = t
        _accum_stats(st_ref, t)

    H2 = 2 * Hd
    t0, st0 = pl.pallas_call(
        k_fc0, grid=(GN,),
        in_specs=[_row_spec(RBN, Hd), _row_spec(RBN, Hd), _row_spec(RBN, Hd),
                  _full_spec((8, Hd)), _full_spec((1, Hd)), _full_spec((1, Hd)),
                  _full_spec((Hd, H2)), _full_spec((1, H2))],
        out_specs=[_row_spec(RBN, H2), _full_spec((8, H2))],
        out_shape=[jax.ShapeDtypeStruct((N, H2), f32),
                   jax.ShapeDtypeStruct((8, H2), f32)],
        name="tc_fc0")(x_inter2, x_intra, t_m, st_m,
                       p['gin4_bn_g'].reshape(1, -1),
                       p['gin4_bn_B'].reshape(1, -1),
                       p['fc0_W'], b_row('fc0'))

    def k_fcmid(t_ref, st, g, bb, w, b, t2_ref, st2_ref):
        x1 = _bn_apply(t_ref[...], st[...], g[...], bb[...], float(N))
        t = _lrelu(_mm(x1, w[...]) + b[...])
        t2_ref[...] = t
        _accum_stats(st2_ref, t)

    def fcmid_call(t, st, bnname, wname, din, dout):
        return pl.pallas_call(
            k_fcmid, grid=(GN,),
            in_specs=[_row_spec(RBN, din), _full_spec((8, din)),
                      _full_spec((1, din)), _full_spec((1, din)),
                      _full_spec((din, dout)), _full_spec((1, dout))],
            out_specs=[_row_spec(RBN, dout), _full_spec((8, dout))],
            out_shape=[jax.ShapeDtypeStruct((N, dout), f32),
                       jax.ShapeDtypeStruct((8, dout), f32)],
            name="tc_" + wname)(t, st, p[bnname + '_g'].reshape(1, -1),
                                p[bnname + '_B'].reshape(1, -1),
                                p[wname + '_W'], b_row(wname))

    t1, st1 = fcmid_call(t0, st0, 'fc0_bn', 'fc1', H2, H2)
    t2, st2 = fcmid_call(t1, st1, 'fc1_bn', 'fc2', H2, Hd)

    def k_fc3(t_ref, st, g, bb, w, b, out_ref):
        x1 = _bn_apply(t_ref[...], st[...], g[...], bb[...], float(N))
        out_ref[...] = _mm(x1, w[...]) + b[...]

    H_upd = pl.pallas_call(
        k_fc3, grid=(GN,),
        in_specs=[_row_spec(RBN, Hd), _full_spec((8, Hd)),
                  _full_spec((1, Hd)), _full_spec((1, Hd)),
                  _full_spec((Hd, Hd)), _full_spec((1, Hd))],
        out_specs=_row_spec(RBN, Hd),
        out_shape=jax.ShapeDtypeStruct((N, Hd), f32),
        name="tc_fc3")(t2, st2, p['fc2_bn_g'].reshape(1, -1),
                       p['fc2_bn_B'].reshape(1, -1),
                       p['fc3_W'], b_row('fc3'))

    NPAD = 10240
    hu_pad = jnp.pad(H_upd, ((0, NPAD - N), (0, 0)))
    blk_pad = jnp.pad(blk_id, (0, NPAD - N))
    blk_parts = _sc_segsum(hu_pad, blk_pad, NB, 80,
                           jnp.zeros((NB, Hd), f32), "sc_seg_block")

    RBB = 400
    GB = NB // RBB

    def k_norm2(p0_ref, p1_ref, out_ref):
        v = p0_ref[...] + p1_ref[...]
        nrm = jnp.sqrt(jnp.sum(v * v, axis=1, keepdims=True))
        out_ref[...] = v / jnp.maximum(nrm, 1e-12)

    block_repr = pl.pallas_call(
        k_norm2, grid=(GB,),
        in_specs=_part_specs(RBB, Hd, NB),
        out_specs=_row_spec(RBB, Hd),
        out_shape=jax.ShapeDtypeStruct((NB, Hd), f32),
        name="tc_blocknorm")(blk_parts, blk_parts)

    NBPAD = 2048
    br_pad = jnp.pad(block_repr, ((0, NBPAD - NB), (0, 0)))
    bat_pad = jnp.pad(bat_id, (0, NBPAD - NB))
    bat_parts = _sc_segsum(br_pad, bat_pad, BS, 64,
                           jnp.zeros((BS, Hd), f32), "sc_seg_batch")

    graph_repr = pl.pallas_call(
        k_norm2, grid=(1,),
        in_specs=_part_specs(BS, Hd, BS),
        out_specs=_row_spec(BS, Hd),
        out_shape=jax.ShapeDtypeStruct((BS, Hd), f32),
        name="tc_graphnorm")(bat_parts, bat_parts)

    return (H_upd, block_repr, graph_repr, Z)
